# baseline (speedup 1.0000x reference)
import functools

import numpy as np

import jax
import jax.numpy as jnp
from jax import lax
from jax.experimental import pallas as pl
from jax.experimental.pallas import tpu as pltpu
from jax.experimental.pallas import tpu_sc as plsc

D = 200
W = 128
S = 25
K = 3
RW = 80
RW2 = 2 * RW
B = 4096
NC, NS, L = 2, 16, 16
NW = NC * NS
PER_W = B // NW
NCH = -(-D // (2 * L))


def _sc_scores(tpk, wpad, lists):
    mesh = plsc.VectorSubcoreMesh(core_axis_name="c", subcore_axis_name="s")

    @functools.partial(
        pl.kernel,
        out_type=jax.ShapeDtypeStruct((NW, PER_W), jnp.float32),
        mesh=mesh,
        compiler_params=pltpu.CompilerParams(use_tc_tiling_on_sc=False,
                                             needs_layout_passes=False),
        scratch_types=[
            pltpu.VMEM((L,), jnp.float32),
            pltpu.VMEM((PER_W * RW2,), jnp.int32),
            pltpu.VMEM((RW2, W), jnp.float32),
            pltpu.VMEM((RW2, W), jnp.float32),
            pltpu.VMEM((PER_W,), jnp.float32),
            pltpu.SemaphoreType.DMA,
            pltpu.SemaphoreType.DMA,
        ],
    )
    def body(tpk_hbm, w_hbm, lists_hbm, out_hbm, w_v, idx_v, rows_a, rows_b,
             out_v, sem_a, sem_b):
        wid = lax.axis_index("s") * NC + lax.axis_index("c")
        pltpu.sync_copy(w_hbm, w_v)
        pltpu.sync_copy(lists_hbm.at[pl.ds(wid * PER_W * RW2, PER_W * RW2)],
                        idx_v)

        lanes = lax.iota(jnp.int32, L)
        zero = jnp.zeros((L,), jnp.float32)

        wv = w_v[...]
        e = jnp.exp(wv - jnp.max(wv))
        wn = e / jnp.full((L,), jnp.sum(e), jnp.float32)
        w0 = jnp.sum(jnp.where(lanes == 0, wn, zero))
        w1 = jnp.sum(jnp.where(lanes == 1, wn, zero)) * (1.0 / S)
        w2 = jnp.sum(jnp.where(lanes == 2, wn, zero)) * (1.0 / S)
        w3 = jnp.sum(jnp.where(lanes == 3, wn, zero)) * (1.0 / S)

        def issue(n, rows_v, sem):
            pltpu.async_copy(
                tpk_hbm.at[idx_v.at[pl.ds(n * RW2, RW2)]], rows_v, sem)

        def drain(rows_v, sem):
            pltpu.make_async_copy(tpk_hbm.at[pl.ds(0, RW2)], rows_v,
                                  sem).wait()

        def row_chunks(rows_v, base, j):
            out = []
            for c in range(NCH):
                bv = rows_v[base + j, pl.ds(c * L, L)]
                out.append(plsc.bitcast(bv, jnp.int32))
            return out

        def unpack_acc(accs, chunks):
            res = list(accs)
            for i, v in enumerate(chunks):
                eo = (lax.bitcast_convert_type(lax.shift_left(v, 16),
                                               jnp.float32),
                      lax.bitcast_convert_type(
                          jnp.bitwise_and(v, jnp.int32(-65536)), jnp.float32))
                for h in range(2):
                    res[2 * i + h] = res[2 * i + h] + eo[h]
            return res

        NACC = 2 * NCH

        def side_repr(rows_v, base):
            e0 = unpack_acc([zero] * NACC, row_chunks(rows_v, base, 0))

            def group(first):
                def gbody(r, accs):
                    return tuple(unpack_acc(accs,
                                            row_chunks(rows_v, base,
                                                       first + r)))
                return lax.fori_loop(0, S, gbody, tuple([zero] * NACC))

            g1 = group(1)
            g2 = group(1 + S)
            g3 = group(1 + 2 * S)
            return [w0 * a + w1 * b + w2 * c + w3 * d
                    for a, b, c, d in zip(e0, g1, g2, g3)]

        def compute(n, rows_v):
            u = side_repr(rows_v, 0)
            v = side_repr(rows_v, RW)
            p = u[0] * v[0]
            for c in range(1, NACC):
                p = p + u[c] * v[c]
            dot = jnp.sum(p)
            plsc.store_scatter(out_v, [jnp.full((L,), n, jnp.int32)],
                               jnp.full((L,), dot, jnp.float32),
                               mask=lanes == 0)

        issue(0, rows_a, sem_a)
        issue(1, rows_b, sem_b)

        def grp(g, carry):
            for n, rows_v, sem in ((2 * g, rows_a, sem_a),
                                   (2 * g + 1, rows_b, sem_b)):
                drain(rows_v, sem)
                compute(n, rows_v)

                @pl.when(n + 2 < PER_W)
                def _():
                    issue(n + 2, rows_v, sem)
            return carry

        lax.fori_loop(0, PER_W // 2, grp, 0)
        pltpu.sync_copy(out_v, out_hbm.at[wid])

    return body(tpk, wpad, lists)


def _pack_lists(ind, walks):
    wt = walks.transpose(0, 2, 1)
    w75 = wt[:, 1:, :].reshape(B, S * K)
    pad = wt[:, 0, :4]
    return jnp.concatenate([ind[:, None], w75, pad], axis=1)


def kernel(node_embeddings, user_weights, item_weights,
           user_indices, item_indices, user_walks, item_walks):
    del item_weights
    wpad = jnp.pad(user_weights, (0, L - user_weights.shape[0]),
                   constant_values=-jnp.inf)
    u = lax.bitcast_convert_type(node_embeddings, jnp.int32)

    def rnd(x):
        return lax.shift_right_logical(
            x + 0x7FFF + (lax.shift_right_logical(x, 16) & 1), 16)

    packed = rnd(u[:, 0::2]) | lax.shift_left(rnd(u[:, 1::2]), 16)
    packed = jnp.pad(packed, ((0, 0), (0, W - D // 2)))
    tpk = lax.bitcast_convert_type(packed, jnp.float32)
    ulists = _pack_lists(user_indices.astype(jnp.int32),
                         user_walks.astype(jnp.int32))
    ilists = _pack_lists(item_indices.astype(jnp.int32),
                         item_walks.astype(jnp.int32))
    lists = jnp.concatenate([ulists, ilists], axis=1).reshape(-1)
    out = _sc_scores(tpk, wpad, lists)
    return out.reshape(B)


# speedup vs baseline: 4.4961x; 4.4961x over previous
import functools

import numpy as np

import jax
import jax.numpy as jnp
from jax import lax
from jax.experimental import pallas as pl
from jax.experimental.pallas import tpu as pltpu
from jax.experimental.pallas import tpu_sc as plsc

D = 200
W = 128
S = 25
K = 3
RW = 80
RW2 = 2 * RW
B = 4096
NC, NS, L = 2, 16, 16
NW = NC * NS
PER_W = B // NW
NCH = -(-D // (2 * L))


def _sc_scores(tpk, wpad, lists):
    mesh = plsc.VectorSubcoreMesh(core_axis_name="c", subcore_axis_name="s")

    @functools.partial(
        pl.kernel,
        out_type=jax.ShapeDtypeStruct((NW, PER_W), jnp.float32),
        mesh=mesh,
        compiler_params=pltpu.CompilerParams(use_tc_tiling_on_sc=False,
                                             needs_layout_passes=False),
        scratch_types=[
            pltpu.VMEM((L,), jnp.float32),
            pltpu.VMEM((PER_W * RW2,), jnp.int32),
            pltpu.VMEM((RW2, W), jnp.float32),
            pltpu.VMEM((RW2, W), jnp.float32),
            pltpu.VMEM((PER_W,), jnp.float32),
            pltpu.SemaphoreType.DMA,
            pltpu.SemaphoreType.DMA,
        ],
    )
    def body(tpk_hbm, w_hbm, lists_hbm, out_hbm, w_v, idx_v, rows_a, rows_b,
             out_v, sem_a, sem_b):
        wid = lax.axis_index("s") * NC + lax.axis_index("c")
        pltpu.sync_copy(w_hbm, w_v)
        pltpu.sync_copy(lists_hbm.at[pl.ds(wid * PER_W * RW2, PER_W * RW2)],
                        idx_v)

        lanes = lax.iota(jnp.int32, L)
        zero = jnp.zeros((L,), jnp.float32)

        wv = w_v[...]
        e = jnp.exp(wv - jnp.max(wv))
        wn = e / jnp.full((L,), jnp.sum(e), jnp.float32)
        w0 = jnp.sum(jnp.where(lanes == 0, wn, zero))
        w1 = jnp.sum(jnp.where(lanes == 1, wn, zero)) * (1.0 / S)
        w2 = jnp.sum(jnp.where(lanes == 2, wn, zero)) * (1.0 / S)
        w3 = jnp.sum(jnp.where(lanes == 3, wn, zero)) * (1.0 / S)

        def issue(n, rows_v, sem):
            pltpu.async_copy(
                tpk_hbm.at[idx_v.at[pl.ds(n * RW2, RW2)]], rows_v, sem)

        def drain(rows_v, sem):
            pltpu.make_async_copy(tpk_hbm.at[pl.ds(0, RW2)], rows_v,
                                  sem).wait()

        def row_chunks(rows_v, base, j):
            out = []
            for c in range(NCH):
                bv = rows_v[base + j, pl.ds(c * L, L)]
                out.append(plsc.bitcast(bv, jnp.int32))
            return out

        def unpack_acc(accs, chunks):
            res = list(accs)
            for i, v in enumerate(chunks):
                eo = (lax.bitcast_convert_type(lax.shift_left(v, 16),
                                               jnp.float32),
                      lax.bitcast_convert_type(
                          jnp.bitwise_and(v, jnp.int32(-65536)), jnp.float32))
                for h in range(2):
                    res[2 * i + h] = res[2 * i + h] + eo[h]
            return res

        NACC = 2 * NCH

        def side_repr(rows_v, base):
            e0 = unpack_acc([zero] * NACC, row_chunks(rows_v, base, 0))

            def group(first):
                def gbody(r, accs):
                    return tuple(unpack_acc(accs,
                                            row_chunks(rows_v, base,
                                                       first + r)))
                return lax.fori_loop(0, S, gbody, tuple([zero] * NACC))

            g1 = group(1)
            g2 = group(1 + S)
            g3 = group(1 + 2 * S)
            return [w0 * a + w1 * b + w2 * c + w3 * d
                    for a, b, c, d in zip(e0, g1, g2, g3)]

        def compute(n, rows_v):
            u = side_repr(rows_v, 0)
            v = side_repr(rows_v, RW)
            p = u[0] * v[0]
            for c in range(1, NACC):
                p = p + u[c] * v[c]
            dot = jnp.sum(p)
            plsc.store_scatter(out_v, [jnp.full((L,), n, jnp.int32)],
                               jnp.full((L,), dot, jnp.float32),
                               mask=lanes == 0)

        issue(0, rows_a, sem_a)
        issue(1, rows_b, sem_b)

        def grp(g, carry):
            for n, rows_v, sem in ((2 * g, rows_a, sem_a),
                                   (2 * g + 1, rows_b, sem_b)):
                drain(rows_v, sem)
                compute(n, rows_v)

                @pl.when(n + 2 < PER_W)
                def _():
                    issue(n + 2, rows_v, sem)
            return carry

        lax.fori_loop(0, PER_W // 2, grp, 0)
        pltpu.sync_copy(out_v, out_hbm.at[wid])

    return body(tpk, wpad, lists)


def _pack_lists(ind, walks):
    wt = walks.transpose(0, 2, 1)
    w75 = wt[:, 1:, :].reshape(B, S * K)
    pad = wt[:, 0, :4]
    return jnp.concatenate([ind[:, None], w75, pad], axis=1)


def kernel(node_embeddings, user_weights, item_weights,
           user_indices, item_indices, user_walks, item_walks):
    del item_weights
    wpad = jnp.pad(user_weights, (0, L - user_weights.shape[0]),
                   constant_values=-jnp.inf)
    u = lax.bitcast_convert_type(node_embeddings, jnp.int32)

    def rnd(x):
        return lax.shift_right_logical(
            x + 0x7FFF + (lax.shift_right_logical(x, 16) & 1), 16)

    packed = rnd(u[:, :D // 2]) | lax.shift_left(rnd(u[:, D // 2:]), 16)
    packed = jnp.pad(packed, ((0, 0), (0, W - D // 2)))
    tpk = lax.bitcast_convert_type(packed, jnp.float32)
    ulists = _pack_lists(user_indices.astype(jnp.int32),
                         user_walks.astype(jnp.int32))
    ilists = _pack_lists(item_indices.astype(jnp.int32),
                         item_walks.astype(jnp.int32))
    lists = jnp.concatenate([ulists, ilists], axis=1).reshape(-1)
    out = _sc_scores(tpk, wpad, lists)
    return out.reshape(B)


# speedup vs baseline: 7.7031x; 1.7133x over previous
import functools

import numpy as np

import jax
import jax.numpy as jnp
from jax import lax
from jax.experimental import pallas as pl
from jax.experimental.pallas import tpu as pltpu
from jax.experimental.pallas import tpu_sc as plsc

D = 200
W = 128
S = 25
K = 3
RW = 80
RW2 = 2 * RW
B = 4096
NC, NS, L = 2, 16, 16
NW = NC * NS
PER_W = B // NW
NCH = -(-D // (2 * L))


def _sc_scores(tpk, wpad, lists):
    mesh = plsc.VectorSubcoreMesh(core_axis_name="c", subcore_axis_name="s")

    @functools.partial(
        pl.kernel,
        out_type=jax.ShapeDtypeStruct((NW, PER_W), jnp.float32),
        mesh=mesh,
        compiler_params=pltpu.CompilerParams(use_tc_tiling_on_sc=False,
                                             needs_layout_passes=False),
        scratch_types=[
            pltpu.VMEM((L,), jnp.float32),
            pltpu.VMEM((2 * PER_W, W), jnp.int32),
            pltpu.VMEM((RW2, W), jnp.float32),
            pltpu.VMEM((RW2, W), jnp.float32),
            pltpu.VMEM((PER_W,), jnp.float32),
            pltpu.SemaphoreType.DMA,
            pltpu.SemaphoreType.DMA,
        ],
    )
    def body(tpk_hbm, w_hbm, lists_hbm, out_hbm, w_v, idx_v, rows_a, rows_b,
             out_v, sem_a, sem_b):
        wid = lax.axis_index("s") * NC + lax.axis_index("c")
        pltpu.sync_copy(w_hbm, w_v)
        pltpu.sync_copy(lists_hbm.at[pl.ds(wid * 2 * PER_W, 2 * PER_W)],
                        idx_v)

        lanes = lax.iota(jnp.int32, L)
        zero = jnp.zeros((L,), jnp.float32)

        wv = w_v[...]
        e = jnp.exp(wv - jnp.max(wv))
        wn = e / jnp.full((L,), jnp.sum(e), jnp.float32)
        w0 = jnp.sum(jnp.where(lanes == 0, wn, zero))
        w1 = jnp.sum(jnp.where(lanes == 1, wn, zero)) * (1.0 / S)
        w2 = jnp.sum(jnp.where(lanes == 2, wn, zero)) * (1.0 / S)
        w3 = jnp.sum(jnp.where(lanes == 3, wn, zero)) * (1.0 / S)

        def issue(n, rows_v, sem):
            pltpu.async_copy(tpk_hbm.at[idx_v.at[2 * n]],
                             rows_v.at[pl.ds(0, W)], sem)
            pltpu.async_copy(tpk_hbm.at[idx_v.at[2 * n + 1, pl.ds(0, RW2 - W)]],
                             rows_v.at[pl.ds(W, RW2 - W)], sem)

        def drain(rows_v, sem):
            pltpu.make_async_copy(tpk_hbm.at[pl.ds(0, RW2)], rows_v,
                                  sem).wait()

        def row_chunks(rows_v, base, j):
            out = []
            for c in range(NCH):
                bv = rows_v[base + j, pl.ds(c * L, L)]
                out.append(plsc.bitcast(bv, jnp.int32))
            return out

        def unpack_acc(accs, chunks):
            res = list(accs)
            for i, v in enumerate(chunks):
                eo = (lax.bitcast_convert_type(lax.shift_left(v, 16),
                                               jnp.float32),
                      lax.bitcast_convert_type(
                          jnp.bitwise_and(v, jnp.int32(-65536)), jnp.float32))
                for h in range(2):
                    res[2 * i + h] = res[2 * i + h] + eo[h]
            return res

        NACC = 2 * NCH

        def side_repr(rows_v, base):
            e0 = unpack_acc([zero] * NACC, row_chunks(rows_v, base, 0))

            def group(first):
                def gbody(r, accs):
                    return tuple(unpack_acc(accs,
                                            row_chunks(rows_v, base,
                                                       first + r)))
                return lax.fori_loop(0, S, gbody, tuple([zero] * NACC))

            g1 = group(1)
            g2 = group(1 + S)
            g3 = group(1 + 2 * S)
            return [w0 * a + w1 * b + w2 * c + w3 * d
                    for a, b, c, d in zip(e0, g1, g2, g3)]

        def compute(n, rows_v):
            u = side_repr(rows_v, 0)
            v = side_repr(rows_v, RW)
            p = u[0] * v[0]
            for c in range(1, NACC):
                p = p + u[c] * v[c]
            dot = jnp.sum(p)
            plsc.store_scatter(out_v, [jnp.full((L,), n, jnp.int32)],
                               jnp.full((L,), dot, jnp.float32),
                               mask=lanes == 0)

        issue(0, rows_a, sem_a)
        issue(1, rows_b, sem_b)

        def grp(g, carry):
            for n, rows_v, sem in ((2 * g, rows_a, sem_a),
                                   (2 * g + 1, rows_b, sem_b)):
                drain(rows_v, sem)
                compute(n, rows_v)

                @pl.when(n + 2 < PER_W)
                def _():
                    issue(n + 2, rows_v, sem)
            return carry

        lax.fori_loop(0, PER_W // 2, grp, 0)
        pltpu.sync_copy(out_v, out_hbm.at[wid])

    return body(tpk, wpad, lists)


def _pack_lists(ind, walks):
    wt = walks.transpose(0, 2, 1)
    w75 = wt[:, 1:, :].reshape(B, S * K)
    pad = wt[:, 0, :4]
    return jnp.concatenate([ind[:, None], w75, pad], axis=1)


def kernel(node_embeddings, user_weights, item_weights,
           user_indices, item_indices, user_walks, item_walks):
    del item_weights
    wpad = jnp.pad(user_weights, (0, L - user_weights.shape[0]),
                   constant_values=-jnp.inf)
    def rnd(half):
        x = lax.bitcast_convert_type(half, jnp.int32)
        return lax.shift_right_logical(
            x + 0x7FFF + (lax.shift_right_logical(x, 16) & 1), 16)

    packed = rnd(node_embeddings[:, :D // 2]) | lax.shift_left(
        rnd(node_embeddings[:, D // 2:]), 16)
    packed = jnp.pad(packed, ((0, 0), (0, W - D // 2)))
    tpk = lax.bitcast_convert_type(packed, jnp.float32)
    ulists = _pack_lists(user_indices.astype(jnp.int32),
                         user_walks.astype(jnp.int32))
    ilists = _pack_lists(item_indices.astype(jnp.int32),
                         item_walks.astype(jnp.int32))
    zpad = jnp.zeros((B, 2 * W - RW2), jnp.int32)
    lists = jnp.concatenate([ulists, ilists, zpad], axis=1).reshape(-1, W)
    out = _sc_scores(tpk, wpad, lists)
    return out.reshape(B)
